# Initial kernel scaffold; baseline (speedup 1.0000x reference)
import jax
import jax.numpy as jnp
from jax.experimental import pallas as pl


def kernel(segmentation_molecule_left, atom_left, bond_left, inner_segmentation_index_left, inner_index_left, outer_segmentation_index_left, outer_index_left, segmentation_molecule_right, atom_right, bond_right, inner_segmentation_index_right, inner_index_right, outer_segmentation_index_right, outer_index_right, mp_node_W, mp_edge_W1, mp_edge_b1, mp_edge_W2, mp_edge_b2, att_key_W, att_value_W, att_out_W, att_out_b, ln_gamma, ln_beta):
    raise NotImplementedError("write your pallas kernel here")


# speedup vs baseline: 1.9271x; 1.9271x over previous
import functools

import jax
import jax.numpy as jnp
from jax.experimental import pallas as pl
from jax.experimental.pallas import tpu as pltpu

H = 128
TEMP = float(jnp.sqrt(128.0))

_EDGE_BLOCK = 1024


def _leaky(x):
    return jnp.maximum(x, 0.01 * x)


def _edge_proj_body(bond_ref, w1t_ref, b1_ref, w2t_ref, b2_ref, out_ref):
    x = bond_ref[...]
    h = _leaky(jnp.dot(x, w1t_ref[...], preferred_element_type=jnp.float32)
               + b1_ref[...])
    y = _leaky(jnp.dot(h, w2t_ref[...], preferred_element_type=jnp.float32)
               + b2_ref[...])
    out_ref[...] = y


def _edge_proj(bond, w1t, b1, w2t, b2):
    n_edges = bond.shape[0]
    grid = (n_edges // _EDGE_BLOCK,)
    return pl.pallas_call(
        _edge_proj_body,
        grid=grid,
        in_specs=[
            pl.BlockSpec((_EDGE_BLOCK, H), lambda i: (i, 0)),
            pl.BlockSpec((H, H), lambda i: (0, 0)),
            pl.BlockSpec((1, H), lambda i: (0, 0)),
            pl.BlockSpec((H, H), lambda i: (0, 0)),
            pl.BlockSpec((1, H), lambda i: (0, 0)),
        ],
        out_specs=pl.BlockSpec((_EDGE_BLOCK, H), lambda i: (i, 0)),
        out_shape=jax.ShapeDtypeStruct((n_edges, H), jnp.float32),
    )(bond, w1t, b1, w2t, b2)


def _segment_softmax_aggregate(t, n_seg, seg, neighbor):
    e = jnp.exp(t / TEMP)
    norm = jax.ops.segment_sum(e, seg, num_segments=n_seg) + 1e-16
    acc = jax.ops.segment_sum(e[:, None] * neighbor, seg, num_segments=n_seg)
    return acc / norm[:, None]


def kernel(segmentation_molecule_left, atom_left, bond_left,
           inner_segmentation_index_left, inner_index_left,
           outer_segmentation_index_left, outer_index_left,
           segmentation_molecule_right, atom_right, bond_right,
           inner_segmentation_index_right, inner_index_right,
           outer_segmentation_index_right, outer_index_right,
           mp_node_W, mp_edge_W1, mp_edge_b1, mp_edge_W2, mp_edge_b2,
           att_key_W, att_value_W, att_out_W, att_out_b, ln_gamma, ln_beta):
    n_nodes = atom_left.shape[0]
    n_mols = 256

    w1t = mp_edge_W1.T
    w2t = mp_edge_W2.T
    b1 = mp_edge_b1[None, :]
    b2 = mp_edge_b2[None, :]

    e_l = _edge_proj(bond_left, w1t, b1, w2t, b2)
    e_r = _edge_proj(bond_right, w1t, b1, w2t, b2)

    k_l = atom_left @ att_key_W.T
    k_r = atom_right @ att_key_W.T
    v_l = atom_left @ att_value_W.T
    v_r = atom_right @ att_value_W.T
    m_l = atom_left @ mp_node_W.T
    m_r = atom_right @ mp_node_W.T

    osl = outer_segmentation_index_left
    osr = outer_segmentation_index_right
    t = (k_l[osl] * k_r[osr]).sum(1)
    ml_msg = _segment_softmax_aggregate(t, n_nodes, osl, v_r[osr])
    mr_msg = _segment_softmax_aggregate(t, n_nodes, osr, v_l[osl])
    outer_l = _leaky(ml_msg @ att_out_W.T + att_out_b)
    outer_r = _leaky(mr_msg @ att_out_W.T + att_out_b)

    inner_l = jnp.zeros_like(atom_left).at[inner_segmentation_index_left].add(
        m_l[inner_index_left] * e_l)
    inner_r = jnp.zeros_like(atom_right).at[inner_segmentation_index_right].add(
        m_r[inner_index_right] * e_r)

    def _ln(x):
        m = x.mean(-1, keepdims=True)
        v = ((x - m) ** 2).mean(-1, keepdims=True)
        return (x - m) / jnp.sqrt(v + 1e-5) * ln_gamma + ln_beta

    msg_l = _ln(atom_left + inner_l + outer_l)
    msg_r = _ln(atom_right + inner_r + outer_r)
    graph_l = jax.ops.segment_sum(msg_l, segmentation_molecule_left,
                                  num_segments=n_mols)
    graph_r = jax.ops.segment_sum(msg_r, segmentation_molecule_right,
                                  num_segments=n_mols)
    return (graph_l, graph_r)


# speedup vs baseline: 5.4246x; 2.8148x over previous
import functools
import math

import jax
import jax.numpy as jnp
from jax import lax
from jax.experimental import pallas as pl
from jax.experimental.pallas import tpu as pltpu
from jax.experimental.pallas import tpu_sc as plsc

H = 128
TEMP = math.sqrt(128.0)

_NC = 2
_NS = 16
_NW = _NC * _NS
_UNIT = 128

_EDGE_BLOCK = 640


def _leaky(x):
    return jnp.maximum(x, 0.01 * x)


def _edge_proj_body(bond_ref, w1t_ref, b1_ref, w2t_ref, b2_ref, out_ref):
    x = bond_ref[...]
    h = _leaky(jnp.dot(x, w1t_ref[...], preferred_element_type=jnp.float32)
               + b1_ref[...])
    y = _leaky(jnp.dot(h, w2t_ref[...], preferred_element_type=jnp.float32)
               + b2_ref[...])
    out_ref[...] = y


def _edge_proj(bond, w1t, b1, w2t, b2):
    n_edges = bond.shape[0]
    grid = (n_edges // _EDGE_BLOCK,)
    return pl.pallas_call(
        _edge_proj_body,
        grid=grid,
        in_specs=[
            pl.BlockSpec((_EDGE_BLOCK, H), lambda i: (i, 0)),
            pl.BlockSpec((H, H), lambda i: (0, 0)),
            pl.BlockSpec((1, H), lambda i: (0, 0)),
            pl.BlockSpec((H, H), lambda i: (0, 0)),
            pl.BlockSpec((1, H), lambda i: (0, 0)),
        ],
        out_specs=pl.BlockSpec((_EDGE_BLOCK, H), lambda i: (i, 0)),
        out_shape=jax.ShapeDtypeStruct((n_edges, H), jnp.float32),
    )(bond, w1t, b1, w2t, b2)


def _zero_own_slice(acc_sh, zb_v, sid, rows_per_sub, zrows):
    def zb_body(i, _):
        for d in range(8):
            zb_v[i, pl.ds(d * 16, 16)] = jnp.zeros((16,), jnp.float32)
        return 0
    lax.fori_loop(0, zrows, zb_body, 0)
    for k in range(rows_per_sub // zrows):
        pltpu.sync_copy(zb_v, acc_sh.at[pl.ds(sid * rows_per_sub + k * zrows,
                                              zrows)])


def _dump_own_slice(acc_sh, zb_v, out_hbm, cid, sid, rows_per_sub, zrows):
    for k in range(rows_per_sub // zrows):
        r0 = sid * rows_per_sub + k * zrows
        pltpu.sync_copy(acc_sh.at[pl.ds(r0, zrows)], zb_v)
        pltpu.sync_copy(zb_v, out_hbm.at[cid, pl.ds(r0, zrows)])


def _sc_inner_phase(m_hbm, e_hbm, idx_hbm, seg_hbm, out_hbm,
                    idx_v, seg_v, rows_v, ef_v, acc_sh, gsem,
                    cid, sid, wid, n_units, rows_per_sub, zrows):
    _zero_own_slice(acc_sh, rows_v, sid, rows_per_sub, zrows)
    plsc.subcore_barrier()

    n_mine = (n_units // _NW) + jnp.where(wid < (n_units % _NW), 1, 0)

    def unit_body(j, _):
        base = (wid + j * _NW) * _UNIT
        pltpu.sync_copy(idx_hbm.at[pl.ds(base, _UNIT)], idx_v)
        pltpu.sync_copy(seg_hbm.at[pl.ds(base, _UNIT)], seg_v)
        g = pltpu.async_copy(m_hbm.at[idx_v], rows_v, gsem)
        pltpu.sync_copy(e_hbm.at[pl.ds(base, _UNIT)], ef_v)
        g.wait()

        def mul_body(i, _):
            for d in range(8):
                sl = pl.ds(d * 16, 16)
                rows_v[i, sl] = rows_v[i, sl] * ef_v[i, sl]
            return 0
        lax.fori_loop(0, _UNIT, mul_body, 0)
        pltpu.sync_copy(rows_v, acc_sh.at[seg_v], add=True)
        return 0

    lax.fori_loop(0, n_mine, unit_body, 0)
    plsc.subcore_barrier()
    _dump_own_slice(acc_sh, rows_v, out_hbm, cid, sid, rows_per_sub, zrows)
    plsc.subcore_barrier()


_NPAD = 10240


def _sc_inner(m_l, e_l, idx_l, seg_l, m_r, e_r, idx_r, seg_r):
    n_edges = e_l.shape[0]
    n_units = n_edges // _UNIT
    rows_per_sub = _NPAD // _NS
    zrows = _UNIT

    mesh = plsc.VectorSubcoreMesh(core_axis_name="c", subcore_axis_name="s")

    @functools.partial(
        pl.kernel,
        mesh=mesh,
        out_type=(jax.ShapeDtypeStruct((_NC, _NPAD, H), jnp.float32),
                  jax.ShapeDtypeStruct((_NC, _NPAD, H), jnp.float32)),
        scratch_types=[
            pltpu.VMEM((_UNIT,), jnp.int32),
            pltpu.VMEM((_UNIT,), jnp.int32),
            pltpu.VMEM((_UNIT, H), jnp.float32),
            pltpu.VMEM((_UNIT, H), jnp.float32),
            pltpu.VMEM_SHARED((_NPAD, H), jnp.float32),
            pltpu.SemaphoreType.DMA,
        ],
        compiler_params=pltpu.CompilerParams(needs_layout_passes=False),
    )
    def k(ml_hbm, el_hbm, il_hbm, sl_hbm, mr_hbm, er_hbm, ir_hbm, sr_hbm,
          outl_hbm, outr_hbm, idx_v, seg_v, rows_v, ef_v, acc_sh, gsem):
        cid = lax.axis_index("c")
        sid = lax.axis_index("s")
        wid = sid * _NC + cid
        _sc_inner_phase(ml_hbm, el_hbm, il_hbm, sl_hbm, outl_hbm,
                        idx_v, seg_v, rows_v, ef_v, acc_sh, gsem,
                        cid, sid, wid, n_units, rows_per_sub, zrows)
        _sc_inner_phase(mr_hbm, er_hbm, ir_hbm, sr_hbm, outr_hbm,
                        idx_v, seg_v, rows_v, ef_v, acc_sh, gsem,
                        cid, sid, wid, n_units, rows_per_sub, zrows)

    return k(m_l, e_l, idx_l, seg_l, m_r, e_r, idx_r, seg_r)


def _att_mul_scatter(vt_hbm, idx_g, idx_s, wbuf_v, rows_v, acc_sh, s_sh, gsem):
    pltpu.async_copy(vt_hbm.at[idx_g], rows_v, gsem).wait()

    def mul_body(g, _):
        wv = wbuf_v[pl.ds(g * 16, 16)]
        for l in range(16):
            e = g * 16 + l
            w = jnp.full((16,), wv[l], jnp.float32)
            for d in range(8):
                sl = pl.ds(d * 16, 16)
                rows_v[e, sl] = rows_v[e, sl] * w
        return 0
    lax.fori_loop(0, _UNIT // 16, mul_body, 0)
    pltpu.sync_copy(rows_v, acc_sh.at[idx_s], add=True)
    pltpu.sync_copy(wbuf_v, s_sh.at[idx_s], add=True)


def _sc_att_phase_l(kl_hbm, kr_hbm, vr_hbm, osl_hbm, osr_hbm, w_hbm,
                    ia_v, ib_v, a_v, b_v, wbuf_v, acc_sh, s_sh, gsem,
                    wid, n_units):
    inv_t = 1.0 / TEMP
    n_mine = (n_units // _NW) + jnp.where(wid < (n_units % _NW), 1, 0)

    def unit_body(j, _):
        base = (wid + j * _NW) * _UNIT
        pltpu.sync_copy(osl_hbm.at[pl.ds(base, _UNIT)], ia_v)
        pltpu.sync_copy(osr_hbm.at[pl.ds(base, _UNIT)], ib_v)
        ga = pltpu.async_copy(kl_hbm.at[ia_v], a_v, gsem)
        gb = pltpu.async_copy(kr_hbm.at[ib_v], b_v, gsem)
        ga.wait()
        gb.wait()

        lanes = lax.iota(jnp.int32, 16)

        def dot_body(g, _):
            wv = jnp.zeros((16,), jnp.float32)
            for l in range(16):
                e = g * 16 + l
                acc = a_v[e, pl.ds(0, 16)] * b_v[e, pl.ds(0, 16)]
                for d in range(1, 8):
                    sl = pl.ds(d * 16, 16)
                    acc = acc + a_v[e, sl] * b_v[e, sl]
                wv = jnp.where(lanes == l, jnp.sum(acc), wv)
            wbuf_v[pl.ds(g * 16, 16)] = jnp.exp(wv * inv_t)
            return 0
        lax.fori_loop(0, _UNIT // 16, dot_body, 0)
        pltpu.sync_copy(wbuf_v, w_hbm.at[pl.ds(base, _UNIT)])

        _att_mul_scatter(vr_hbm, ib_v, ia_v, wbuf_v, a_v, acc_sh, s_sh, gsem)
        return 0

    lax.fori_loop(0, n_mine, unit_body, 0)


def _sc_att_phase_r(vl_hbm, osl_hbm, osr_hbm, w_hbm,
                    ia_v, ib_v, a_v, wbuf_v, acc_sh, s_sh, gsem,
                    wid, n_units):
    n_mine = (n_units // _NW) + jnp.where(wid < (n_units % _NW), 1, 0)

    def unit_body(j, _):
        base = (wid + j * _NW) * _UNIT
        pltpu.sync_copy(osl_hbm.at[pl.ds(base, _UNIT)], ia_v)
        pltpu.sync_copy(osr_hbm.at[pl.ds(base, _UNIT)], ib_v)
        pltpu.sync_copy(w_hbm.at[pl.ds(base, _UNIT)], wbuf_v)
        _att_mul_scatter(vl_hbm, ia_v, ib_v, wbuf_v, a_v, acc_sh, s_sh, gsem)
        return 0

    lax.fori_loop(0, n_mine, unit_body, 0)


def _zero_s(s_sh, szb_v, sid, spr):
    def z_body(i, _):
        szb_v[pl.ds(i * 16, 16)] = jnp.zeros((16,), jnp.float32)
        return 0
    lax.fori_loop(0, spr // 16, z_body, 0)
    pltpu.sync_copy(szb_v, s_sh.at[pl.ds(sid * spr, spr)])


def _dump_s(s_sh, sdump_v, s_hbm, cid, sid):
    @pl.when(sid == 0)
    def _():
        pltpu.sync_copy(s_sh, sdump_v)
        pltpu.sync_copy(sdump_v, s_hbm.at[pl.ds(cid * _NPAD, _NPAD)])


def _sc_attention(k_l, k_r, v_l, v_r, osl, osr):
    n_edges = osl.shape[0]
    n_units = n_edges // _UNIT
    rows_per_sub = _NPAD // _NS
    spr = _NPAD // _NS

    mesh = plsc.VectorSubcoreMesh(core_axis_name="c", subcore_axis_name="s")

    @functools.partial(
        pl.kernel,
        mesh=mesh,
        out_type=(jax.ShapeDtypeStruct((_NC, _NPAD, H), jnp.float32),
                  jax.ShapeDtypeStruct((_NC * _NPAD,), jnp.float32),
                  jax.ShapeDtypeStruct((_NC, _NPAD, H), jnp.float32),
                  jax.ShapeDtypeStruct((_NC * _NPAD,), jnp.float32),
                  jax.ShapeDtypeStruct((n_edges,), jnp.float32)),
        scratch_types=[
            pltpu.VMEM((_UNIT,), jnp.int32),
            pltpu.VMEM((_UNIT,), jnp.int32),
            pltpu.VMEM((_UNIT, H), jnp.float32),
            pltpu.VMEM((_UNIT, H), jnp.float32),
            pltpu.VMEM((_UNIT,), jnp.float32),
            pltpu.VMEM((spr,), jnp.float32),
            pltpu.VMEM((_NPAD,), jnp.float32),
            pltpu.VMEM_SHARED((_NPAD, H), jnp.float32),
            pltpu.VMEM_SHARED((_NPAD,), jnp.float32),
            pltpu.SemaphoreType.DMA,
        ],
        compiler_params=pltpu.CompilerParams(needs_layout_passes=False),
    )
    def k(kl_hbm, kr_hbm, vl_hbm, vr_hbm, osl_hbm, osr_hbm,
          accl_hbm, sl_hbm, accr_hbm, sr_hbm, w_hbm,
          ia_v, ib_v, a_v, b_v, wbuf_v, szb_v, sdump_v, acc_sh, s_sh, gsem):
        cid = lax.axis_index("c")
        sid = lax.axis_index("s")
        wid = sid * _NC + cid

        _zero_own_slice(acc_sh, a_v, sid, rows_per_sub, _UNIT)
        _zero_s(s_sh, szb_v, sid, spr)
        plsc.subcore_barrier()
        _sc_att_phase_l(kl_hbm, kr_hbm, vr_hbm, osl_hbm, osr_hbm, w_hbm,
                        ia_v, ib_v, a_v, b_v, wbuf_v, acc_sh, s_sh, gsem,
                        wid, n_units)
        plsc.subcore_barrier()
        _dump_own_slice(acc_sh, a_v, accl_hbm, cid, sid, rows_per_sub, _UNIT)
        _dump_s(s_sh, sdump_v, sl_hbm, cid, sid)
        plsc.subcore_barrier()
        _zero_own_slice(acc_sh, a_v, sid, rows_per_sub, _UNIT)
        _zero_s(s_sh, szb_v, sid, spr)
        plsc.subcore_barrier()

        _sc_att_phase_r(vl_hbm, osl_hbm, osr_hbm, w_hbm,
                        ia_v, ib_v, a_v, wbuf_v, acc_sh, s_sh, gsem,
                        wid, n_units)
        plsc.subcore_barrier()
        _dump_own_slice(acc_sh, a_v, accr_hbm, cid, sid, rows_per_sub, _UNIT)
        _dump_s(s_sh, sdump_v, sr_hbm, cid, sid)

    return k(k_l, k_r, v_l, v_r, osl, osr)


def kernel(segmentation_molecule_left, atom_left, bond_left,
           inner_segmentation_index_left, inner_index_left,
           outer_segmentation_index_left, outer_index_left,
           segmentation_molecule_right, atom_right, bond_right,
           inner_segmentation_index_right, inner_index_right,
           outer_segmentation_index_right, outer_index_right,
           mp_node_W, mp_edge_W1, mp_edge_b1, mp_edge_W2, mp_edge_b2,
           att_key_W, att_value_W, att_out_W, att_out_b, ln_gamma, ln_beta):
    n_nodes = atom_left.shape[0]
    n_mols = 256

    w1t = mp_edge_W1.T
    w2t = mp_edge_W2.T
    b1 = mp_edge_b1[None, :]
    b2 = mp_edge_b2[None, :]

    e_l = _edge_proj(bond_left, w1t, b1, w2t, b2)
    e_r = _edge_proj(bond_right, w1t, b1, w2t, b2)

    k_l = atom_left @ att_key_W.T
    k_r = atom_right @ att_key_W.T
    v_l = atom_left @ att_value_W.T
    v_r = atom_right @ att_value_W.T
    m_l = atom_left @ mp_node_W.T
    m_r = atom_right @ mp_node_W.T

    osl = outer_segmentation_index_left
    osr = outer_segmentation_index_right
    pal, psl, par, psr, _w = _sc_attention(k_l, k_r, v_l, v_r, osl, osr)
    ml_msg = ((pal[0] + pal[1])[:n_nodes]
              / (psl[:_NPAD] + psl[_NPAD:] + 1e-16)[:n_nodes, None])
    mr_msg = ((par[0] + par[1])[:n_nodes]
              / (psr[:_NPAD] + psr[_NPAD:] + 1e-16)[:n_nodes, None])
    outer_l = _leaky(ml_msg @ att_out_W.T + att_out_b)
    outer_r = _leaky(mr_msg @ att_out_W.T + att_out_b)

    pin_l, pin_r = _sc_inner(
        m_l, e_l, inner_index_left, inner_segmentation_index_left,
        m_r, e_r, inner_index_right, inner_segmentation_index_right)
    inner_l = (pin_l[0] + pin_l[1])[:n_nodes]
    inner_r = (pin_r[0] + pin_r[1])[:n_nodes]

    def _ln(x):
        m = x.mean(-1, keepdims=True)
        v = ((x - m) ** 2).mean(-1, keepdims=True)
        return (x - m) / jnp.sqrt(v + 1e-5) * ln_gamma + ln_beta

    msg_l = _ln(atom_left + inner_l + outer_l)
    msg_r = _ln(atom_right + inner_r + outer_r)
    graph_l = jax.ops.segment_sum(msg_l, segmentation_molecule_left,
                                  num_segments=n_mols)
    graph_r = jax.ops.segment_sum(msg_r, segmentation_molecule_right,
                                  num_segments=n_mols)
    return (graph_l, graph_r)


# speedup vs baseline: 8.1831x; 1.5085x over previous
import functools
import math

import jax
import jax.numpy as jnp
from jax import lax
from jax.experimental import pallas as pl
from jax.experimental.pallas import tpu as pltpu
from jax.experimental.pallas import tpu_sc as plsc

H = 128
TEMP = math.sqrt(128.0)

_NC = 2
_NS = 16
_NW = _NC * _NS
_UNIT = 64

_EDGE_BLOCK = 640


def _leaky(x):
    return jnp.maximum(x, 0.01 * x)


def _edge_proj_body(bond_ref, w1t_ref, b1_ref, w2t_ref, b2_ref, out_ref):
    x = bond_ref[...]
    h = _leaky(jnp.dot(x, w1t_ref[...], preferred_element_type=jnp.float32)
               + b1_ref[...])
    y = _leaky(jnp.dot(h, w2t_ref[...], preferred_element_type=jnp.float32)
               + b2_ref[...])
    out_ref[...] = y


def _edge_proj(bond, w1t, b1, w2t, b2):
    n_edges = bond.shape[0]
    grid = (n_edges // _EDGE_BLOCK,)
    return pl.pallas_call(
        _edge_proj_body,
        grid=grid,
        in_specs=[
            pl.BlockSpec((_EDGE_BLOCK, H), lambda i: (i, 0)),
            pl.BlockSpec((H, H), lambda i: (0, 0)),
            pl.BlockSpec((1, H), lambda i: (0, 0)),
            pl.BlockSpec((H, H), lambda i: (0, 0)),
            pl.BlockSpec((1, H), lambda i: (0, 0)),
        ],
        out_specs=pl.BlockSpec((_EDGE_BLOCK, H), lambda i: (i, 0)),
        out_shape=jax.ShapeDtypeStruct((n_edges, H), jnp.float32),
    )(bond, w1t, b1, w2t, b2)


def _zero_own_slice(acc_sh, zb_v, sid, rows_per_sub, zrows):
    def zb_body(i, _):
        for d in range(8):
            zb_v[i, pl.ds(d * 16, 16)] = jnp.zeros((16,), jnp.float32)
        return 0
    lax.fori_loop(0, zrows, zb_body, 0)
    for k in range(rows_per_sub // zrows):
        pltpu.sync_copy(zb_v, acc_sh.at[pl.ds(sid * rows_per_sub + k * zrows,
                                              zrows)])


def _dump_own_slice(acc_sh, zb_v, out_hbm, cid, sid, rows_per_sub, zrows):
    for k in range(rows_per_sub // zrows):
        r0 = sid * rows_per_sub + k * zrows
        pltpu.sync_copy(acc_sh.at[pl.ds(r0, zrows)], zb_v)
        pltpu.sync_copy(zb_v, out_hbm.at[cid, pl.ds(r0, zrows)])


def _sc_inner_phase(m_hbm, e_hbm, idx_hbm, seg_hbm, out_hbm,
                    idx_v, seg_v, rows_v, ef_v, lsem, gsem, acc_sh,
                    cid, sid, wid, n_units, rows_per_sub, zrows):
    _zero_own_slice(acc_sh, rows_v[0], sid, rows_per_sub, zrows)
    plsc.subcore_barrier()

    n_pairs = n_units // 2
    n_mine = (n_pairs // _NW) + jnp.where(wid < (n_pairs % _NW), 1, 0)

    def issue_l(u, s):
        base = u * _UNIT
        pltpu.async_copy(idx_hbm.at[pl.ds(base, _UNIT)], idx_v[s], lsem[s])
        pltpu.async_copy(seg_hbm.at[pl.ds(base, _UNIT)], seg_v[s], lsem[s])
        pltpu.async_copy(e_hbm.at[pl.ds(base, _UNIT)], ef_v[s], lsem[s])

    def wait_l(s):
        pltpu.make_async_copy(idx_hbm.at[pl.ds(0, _UNIT)], idx_v[s], lsem[s]).wait()
        pltpu.make_async_copy(seg_hbm.at[pl.ds(0, _UNIT)], seg_v[s], lsem[s]).wait()
        pltpu.make_async_copy(e_hbm.at[pl.ds(0, _UNIT)], ef_v[s], lsem[s]).wait()

    def issue_g(s):
        pltpu.async_copy(m_hbm.at[idx_v[s]], rows_v[s], gsem[s])

    def wait_g(s):
        pltpu.make_async_copy(m_hbm.at[pl.ds(0, _UNIT)], rows_v[s], gsem[s]).wait()

    def compute(s):
        def mul_body(i, _):
            for d in range(8):
                sl = pl.ds(d * 16, 16)
                rows_v[s][i, sl] = rows_v[s][i, sl] * ef_v[s][i, sl]
            return 0
        lax.fori_loop(0, _UNIT, mul_body, 0)
        pltpu.sync_copy(rows_v[s], acc_sh.at[seg_v[s]], add=True)

    a0 = 2 * wid
    issue_l(a0, 0)
    issue_l(a0 + 1, 1)
    wait_l(0)
    issue_g(0)

    def pair_body(q, _):
        nxt = jnp.minimum(wid + (q + 1) * _NW, n_pairs - 1)
        a2 = 2 * nxt
        wait_l(1)
        issue_g(1)
        wait_g(0)
        compute(0)
        issue_l(a2, 0)
        wait_g(1)
        compute(1)
        issue_l(a2 + 1, 1)
        wait_l(0)
        issue_g(0)
        return 0

    lax.fori_loop(0, n_mine, pair_body, 0)
    wait_g(0)
    wait_l(1)
    plsc.subcore_barrier()
    _dump_own_slice(acc_sh, rows_v[0], out_hbm, cid, sid, rows_per_sub, zrows)
    plsc.subcore_barrier()


_NPAD = 10240


def _sc_inner(m_l, e_l, idx_l, seg_l, m_r, e_r, idx_r, seg_r):
    n_edges = e_l.shape[0]
    n_units = n_edges // _UNIT
    rows_per_sub = _NPAD // _NS
    zrows = _UNIT

    mesh = plsc.VectorSubcoreMesh(core_axis_name="c", subcore_axis_name="s")

    @functools.partial(
        pl.kernel,
        mesh=mesh,
        out_type=(jax.ShapeDtypeStruct((_NC, _NPAD, H), jnp.float32),
                  jax.ShapeDtypeStruct((_NC, _NPAD, H), jnp.float32)),
        scratch_types=[
            [pltpu.VMEM((_UNIT,), jnp.int32)] * 2,
            [pltpu.VMEM((_UNIT,), jnp.int32)] * 2,
            [pltpu.VMEM((_UNIT, H), jnp.float32)] * 2,
            [pltpu.VMEM((_UNIT, H), jnp.float32)] * 2,
            pltpu.VMEM_SHARED((_NPAD, H), jnp.float32),
            [pltpu.SemaphoreType.DMA] * 2,
            [pltpu.SemaphoreType.DMA] * 2,
        ],
        compiler_params=pltpu.CompilerParams(needs_layout_passes=False),
    )
    def k(ml_hbm, el_hbm, il_hbm, sl_hbm, mr_hbm, er_hbm, ir_hbm, sr_hbm,
          outl_hbm, outr_hbm, idx_v, seg_v, rows_v, ef_v, acc_sh, lsem, gsem):
        cid = lax.axis_index("c")
        sid = lax.axis_index("s")
        wid = sid * _NC + cid
        _sc_inner_phase(ml_hbm, el_hbm, il_hbm, sl_hbm, outl_hbm,
                        idx_v, seg_v, rows_v, ef_v, lsem, gsem, acc_sh,
                        cid, sid, wid, n_units, rows_per_sub, zrows)
        _sc_inner_phase(mr_hbm, er_hbm, ir_hbm, sr_hbm, outr_hbm,
                        idx_v, seg_v, rows_v, ef_v, lsem, gsem, acc_sh,
                        cid, sid, wid, n_units, rows_per_sub, zrows)

    return k(m_l, e_l, idx_l, seg_l, m_r, e_r, idx_r, seg_r)


def _mul_by_w(rows_ref, wbuf_ref):
    lanes = lax.iota(jnp.int32, 16)

    def mul_body(e, _):
        wv = wbuf_ref[pl.ds((e // 16) * 16, 16)]
        w = jnp.full((16,), jnp.sum(jnp.where(lanes == e % 16, wv, 0.0)),
                     jnp.float32)
        for d in range(8):
            sl = pl.ds(d * 16, 16)
            rows_ref[e, sl] = rows_ref[e, sl] * w
        return 0
    lax.fori_loop(0, _UNIT, mul_body, 0)


def _sc_att_phase_l(kl_hbm, kr_hbm, vr_hbm, osl_hbm, osr_hbm, w_hbm,
                    ia_v, ib_v, a_v, b_v, c_v, wbuf_v, acc_sh, s_sh,
                    lsem, gsem, wid, n_units):
    inv_t = 1.0 / TEMP
    n_pairs = n_units // 2
    n_mine = (n_pairs // _NW) + jnp.where(wid < (n_pairs % _NW), 1, 0)

    def issue_l(u, s):
        base = u * _UNIT
        pltpu.async_copy(osl_hbm.at[pl.ds(base, _UNIT)], ia_v[s], lsem[s])
        pltpu.async_copy(osr_hbm.at[pl.ds(base, _UNIT)], ib_v[s], lsem[s])

    def wait_l(s):
        pltpu.make_async_copy(osl_hbm.at[pl.ds(0, _UNIT)], ia_v[s], lsem[s]).wait()
        pltpu.make_async_copy(osr_hbm.at[pl.ds(0, _UNIT)], ib_v[s], lsem[s]).wait()

    def issue_g(s):
        pltpu.async_copy(kl_hbm.at[ia_v[s]], a_v[s], gsem[s])
        pltpu.async_copy(kr_hbm.at[ib_v[s]], b_v[s], gsem[s])

    def wait_g(s):
        pltpu.make_async_copy(kl_hbm.at[pl.ds(0, _UNIT)], a_v[s], gsem[s]).wait()
        pltpu.make_async_copy(kr_hbm.at[pl.ds(0, _UNIT)], b_v[s], gsem[s]).wait()

    lanes = lax.iota(jnp.int32, 16)

    def compute(u, s):
        base = u * _UNIT
        gv = pltpu.async_copy(vr_hbm.at[ib_v[s]], c_v, gsem[s])

        def dot_body(e, _):
            acc = a_v[s][e, pl.ds(0, 16)] * b_v[s][e, pl.ds(0, 16)]
            for d in range(1, 8):
                sl = pl.ds(d * 16, 16)
                acc = acc + a_v[s][e, sl] * b_v[s][e, sl]
            wsl = pl.ds((e // 16) * 16, 16)
            wbuf_v[s][wsl] = jnp.where(lanes == e % 16, jnp.sum(acc),
                                       wbuf_v[s][wsl])
            return 0
        lax.fori_loop(0, _UNIT, dot_body, 0)

        def exp_body(g, _):
            sl = pl.ds(g * 16, 16)
            wbuf_v[s][sl] = jnp.exp(wbuf_v[s][sl] * inv_t)
            return 0
        lax.fori_loop(0, _UNIT // 16, exp_body, 0)
        pltpu.sync_copy(wbuf_v[s], w_hbm.at[pl.ds(base, _UNIT)])
        gv.wait()
        _mul_by_w(c_v, wbuf_v[s])
        pltpu.sync_copy(c_v, acc_sh.at[ia_v[s]], add=True)
        pltpu.sync_copy(wbuf_v[s], s_sh.at[ia_v[s]], add=True)

    a0 = 2 * wid
    issue_l(a0, 0)
    issue_l(a0 + 1, 1)
    wait_l(0)
    issue_g(0)

    def pair_body(q, _):
        a = 2 * (wid + q * _NW)
        a2 = 2 * jnp.minimum(wid + (q + 1) * _NW, n_pairs - 1)
        wait_l(1)
        issue_g(1)
        wait_g(0)
        compute(a, 0)
        issue_l(a2, 0)
        wait_g(1)
        compute(a + 1, 1)
        issue_l(a2 + 1, 1)
        wait_l(0)
        issue_g(0)
        return 0

    lax.fori_loop(0, n_mine, pair_body, 0)
    wait_g(0)
    wait_l(1)


def _sc_att_phase_r(vl_hbm, osl_hbm, osr_hbm, w_hbm,
                    ia_v, ib_v, rows2_v, wbuf_v, acc_sh, s_sh,
                    lsem, gsem, wid, n_units):
    n_pairs = n_units // 2
    n_mine = (n_pairs // _NW) + jnp.where(wid < (n_pairs % _NW), 1, 0)

    def issue_l(u, s):
        base = u * _UNIT
        pltpu.async_copy(osl_hbm.at[pl.ds(base, _UNIT)], ia_v[s], lsem[s])
        pltpu.async_copy(osr_hbm.at[pl.ds(base, _UNIT)], ib_v[s], lsem[s])
        pltpu.async_copy(w_hbm.at[pl.ds(base, _UNIT)], wbuf_v[s], lsem[s])

    def wait_l(s):
        pltpu.make_async_copy(osl_hbm.at[pl.ds(0, _UNIT)], ia_v[s], lsem[s]).wait()
        pltpu.make_async_copy(osr_hbm.at[pl.ds(0, _UNIT)], ib_v[s], lsem[s]).wait()
        pltpu.make_async_copy(w_hbm.at[pl.ds(0, _UNIT)], wbuf_v[s], lsem[s]).wait()

    def issue_g(s):
        pltpu.async_copy(vl_hbm.at[ia_v[s]], rows2_v[s], gsem[s])

    def wait_g(s):
        pltpu.make_async_copy(vl_hbm.at[pl.ds(0, _UNIT)], rows2_v[s], gsem[s]).wait()

    def compute(s):
        _mul_by_w(rows2_v[s], wbuf_v[s])
        pltpu.sync_copy(rows2_v[s], acc_sh.at[ib_v[s]], add=True)
        pltpu.sync_copy(wbuf_v[s], s_sh.at[ib_v[s]], add=True)

    a0 = 2 * wid
    issue_l(a0, 0)
    issue_l(a0 + 1, 1)
    wait_l(0)
    issue_g(0)

    def pair_body(q, _):
        a2 = 2 * jnp.minimum(wid + (q + 1) * _NW, n_pairs - 1)
        wait_l(1)
        issue_g(1)
        wait_g(0)
        compute(0)
        issue_l(a2, 0)
        wait_g(1)
        compute(1)
        issue_l(a2 + 1, 1)
        wait_l(0)
        issue_g(0)
        return 0

    lax.fori_loop(0, n_mine, pair_body, 0)
    wait_g(0)
    wait_l(1)


def _zero_s(s_sh, szb_v, sid, spr):
    def z_body(i, _):
        szb_v[pl.ds(i * 16, 16)] = jnp.zeros((16,), jnp.float32)
        return 0
    lax.fori_loop(0, spr // 16, z_body, 0)
    pltpu.sync_copy(szb_v, s_sh.at[pl.ds(sid * spr, spr)])


def _dump_s(s_sh, szb_v, s_hbm, cid, sid, spr):
    pltpu.sync_copy(s_sh.at[pl.ds(sid * spr, spr)], szb_v)
    pltpu.sync_copy(szb_v, s_hbm.at[pl.ds(cid * _NPAD + sid * spr, spr)])


def _sc_attention_l(k_l, k_r, v_r, osl, osr):
    n_edges = osl.shape[0]
    n_units = n_edges // _UNIT
    rows_per_sub = _NPAD // _NS
    spr = _NPAD // _NS

    mesh = plsc.VectorSubcoreMesh(core_axis_name="c", subcore_axis_name="s")

    @functools.partial(
        pl.kernel,
        mesh=mesh,
        out_type=(jax.ShapeDtypeStruct((_NC, _NPAD, H), jnp.float32),
                  jax.ShapeDtypeStruct((_NC * _NPAD,), jnp.float32),
                  jax.ShapeDtypeStruct((n_edges,), jnp.float32)),
        scratch_types=[
            [pltpu.VMEM((_UNIT,), jnp.int32)] * 2,
            [pltpu.VMEM((_UNIT,), jnp.int32)] * 2,
            [pltpu.VMEM((_UNIT, H), jnp.float32)] * 2,
            [pltpu.VMEM((_UNIT, H), jnp.float32)] * 2,
            pltpu.VMEM((_UNIT, H), jnp.float32),
            [pltpu.VMEM((_UNIT,), jnp.float32)] * 2,
            pltpu.VMEM((spr,), jnp.float32),
            pltpu.VMEM_SHARED((_NPAD, H), jnp.float32),
            pltpu.VMEM_SHARED((_NPAD,), jnp.float32),
            [pltpu.SemaphoreType.DMA] * 2,
            [pltpu.SemaphoreType.DMA] * 2,
        ],
        compiler_params=pltpu.CompilerParams(needs_layout_passes=False),
    )
    def k(kl_hbm, kr_hbm, vr_hbm, osl_hbm, osr_hbm,
          accl_hbm, sl_hbm, w_hbm,
          ia_v, ib_v, a_v, b_v, c_v, wbuf_v, szb_v, acc_sh, s_sh, lsem, gsem):
        cid = lax.axis_index("c")
        sid = lax.axis_index("s")
        wid = sid * _NC + cid
        _zero_own_slice(acc_sh, a_v[0], sid, rows_per_sub, _UNIT)
        _zero_s(s_sh, szb_v, sid, spr)
        plsc.subcore_barrier()
        _sc_att_phase_l(kl_hbm, kr_hbm, vr_hbm, osl_hbm, osr_hbm, w_hbm,
                        ia_v, ib_v, a_v, b_v, c_v, wbuf_v, acc_sh, s_sh,
                        lsem, gsem, wid, n_units)
        plsc.subcore_barrier()
        _dump_own_slice(acc_sh, a_v[0], accl_hbm, cid, sid, rows_per_sub, _UNIT)
        _dump_s(s_sh, szb_v, sl_hbm, cid, sid, spr)

    return k(k_l, k_r, v_r, osl, osr)


def _sc_attention_r(v_l, osl, osr, w):
    n_edges = osl.shape[0]
    n_units = n_edges // _UNIT
    rows_per_sub = _NPAD // _NS
    spr = _NPAD // _NS

    mesh = plsc.VectorSubcoreMesh(core_axis_name="c", subcore_axis_name="s")

    @functools.partial(
        pl.kernel,
        mesh=mesh,
        out_type=(jax.ShapeDtypeStruct((_NC, _NPAD, H), jnp.float32),
                  jax.ShapeDtypeStruct((_NC * _NPAD,), jnp.float32)),
        scratch_types=[
            [pltpu.VMEM((_UNIT,), jnp.int32)] * 2,
            [pltpu.VMEM((_UNIT,), jnp.int32)] * 2,
            [pltpu.VMEM((_UNIT, H), jnp.float32)] * 2,
            [pltpu.VMEM((_UNIT,), jnp.float32)] * 2,
            pltpu.VMEM((spr,), jnp.float32),
            pltpu.VMEM_SHARED((_NPAD, H), jnp.float32),
            pltpu.VMEM_SHARED((_NPAD,), jnp.float32),
            [pltpu.SemaphoreType.DMA] * 2,
            [pltpu.SemaphoreType.DMA] * 2,
        ],
        compiler_params=pltpu.CompilerParams(needs_layout_passes=False),
    )
    def k(vl_hbm, osl_hbm, osr_hbm, w_hbm, accr_hbm, sr_hbm,
          ia_v, ib_v, r_v, wbuf_v, szb_v, acc_sh, s_sh, lsem, gsem):
        cid = lax.axis_index("c")
        sid = lax.axis_index("s")
        wid = sid * _NC + cid
        _zero_own_slice(acc_sh, r_v[0], sid, rows_per_sub, _UNIT)
        _zero_s(s_sh, szb_v, sid, spr)
        plsc.subcore_barrier()
        _sc_att_phase_r(vl_hbm, osl_hbm, osr_hbm, w_hbm,
                        ia_v, ib_v, r_v, wbuf_v, acc_sh, s_sh,
                        lsem, gsem, wid, n_units)
        plsc.subcore_barrier()
        _dump_own_slice(acc_sh, r_v[0], accr_hbm, cid, sid, rows_per_sub, _UNIT)
        _dump_s(s_sh, szb_v, sr_hbm, cid, sid, spr)

    return k(v_l, osl, osr, w)


def kernel(segmentation_molecule_left, atom_left, bond_left,
           inner_segmentation_index_left, inner_index_left,
           outer_segmentation_index_left, outer_index_left,
           segmentation_molecule_right, atom_right, bond_right,
           inner_segmentation_index_right, inner_index_right,
           outer_segmentation_index_right, outer_index_right,
           mp_node_W, mp_edge_W1, mp_edge_b1, mp_edge_W2, mp_edge_b2,
           att_key_W, att_value_W, att_out_W, att_out_b, ln_gamma, ln_beta):
    n_nodes = atom_left.shape[0]
    n_mols = 256

    w1t = mp_edge_W1.T
    w2t = mp_edge_W2.T
    b1 = mp_edge_b1[None, :]
    b2 = mp_edge_b2[None, :]

    e_l = _edge_proj(bond_left, w1t, b1, w2t, b2)
    e_r = _edge_proj(bond_right, w1t, b1, w2t, b2)

    k_l = atom_left @ att_key_W.T
    k_r = atom_right @ att_key_W.T
    v_l = atom_left @ att_value_W.T
    v_r = atom_right @ att_value_W.T
    m_l = atom_left @ mp_node_W.T
    m_r = atom_right @ mp_node_W.T

    osl = outer_segmentation_index_left
    osr = outer_segmentation_index_right
    pal, psl, w_e = _sc_attention_l(k_l, k_r, v_r, osl, osr)
    par, psr = _sc_attention_r(v_l, osl, osr, w_e)
    ml_msg = ((pal[0] + pal[1])[:n_nodes]
              / (psl[:_NPAD] + psl[_NPAD:] + 1e-16)[:n_nodes, None])
    mr_msg = ((par[0] + par[1])[:n_nodes]
              / (psr[:_NPAD] + psr[_NPAD:] + 1e-16)[:n_nodes, None])
    outer_l = _leaky(ml_msg @ att_out_W.T + att_out_b)
    outer_r = _leaky(mr_msg @ att_out_W.T + att_out_b)

    pin_l, pin_r = _sc_inner(
        m_l, e_l, inner_index_left, inner_segmentation_index_left,
        m_r, e_r, inner_index_right, inner_segmentation_index_right)
    inner_l = (pin_l[0] + pin_l[1])[:n_nodes]
    inner_r = (pin_r[0] + pin_r[1])[:n_nodes]

    def _ln(x):
        m = x.mean(-1, keepdims=True)
        v = ((x - m) ** 2).mean(-1, keepdims=True)
        return (x - m) / jnp.sqrt(v + 1e-5) * ln_gamma + ln_beta

    msg_l = _ln(atom_left + inner_l + outer_l)
    msg_r = _ln(atom_right + inner_r + outer_r)
    graph_l = jax.ops.segment_sum(msg_l, segmentation_molecule_left,
                                  num_segments=n_mols)
    graph_r = jax.ops.segment_sum(msg_r, segmentation_molecule_right,
                                  num_segments=n_mols)
    return (graph_l, graph_r)


# speedup vs baseline: 8.7456x; 1.0687x over previous
import functools
import math

import jax
import jax.numpy as jnp
from jax import lax
from jax.experimental import pallas as pl
from jax.experimental.pallas import tpu as pltpu
from jax.experimental.pallas import tpu_sc as plsc

H = 128
TEMP = math.sqrt(128.0)

_NC = 2
_NS = 16
_NW = _NC * _NS
_UNIT = 64

_EDGE_BLOCK = 640


def _leaky(x):
    return jnp.maximum(x, 0.01 * x)


def _edge_proj_body(bond_ref, w1t_ref, b1_ref, w2t_ref, b2_ref, out_ref):
    x = bond_ref[...]
    h = _leaky(jnp.dot(x, w1t_ref[...], preferred_element_type=jnp.float32)
               + b1_ref[...])
    y = _leaky(jnp.dot(h, w2t_ref[...], preferred_element_type=jnp.float32)
               + b2_ref[...])
    out_ref[...] = y


def _edge_proj(bond, w1t, b1, w2t, b2):
    n_edges = bond.shape[0]
    grid = (n_edges // _EDGE_BLOCK,)
    return pl.pallas_call(
        _edge_proj_body,
        grid=grid,
        in_specs=[
            pl.BlockSpec((_EDGE_BLOCK, H), lambda i: (i, 0)),
            pl.BlockSpec((H, H), lambda i: (0, 0)),
            pl.BlockSpec((1, H), lambda i: (0, 0)),
            pl.BlockSpec((H, H), lambda i: (0, 0)),
            pl.BlockSpec((1, H), lambda i: (0, 0)),
        ],
        out_specs=pl.BlockSpec((_EDGE_BLOCK, H), lambda i: (i, 0)),
        out_shape=jax.ShapeDtypeStruct((n_edges, H), jnp.float32),
    )(bond, w1t, b1, w2t, b2)


_NODE_BLOCK = 400


def _node_proj_body(atom_ref, wk_ref, wv_ref, wn_ref, k_ref, v_ref, m_ref):
    x = atom_ref[...]
    k_ref[...] = jnp.dot(x, wk_ref[...], preferred_element_type=jnp.float32)
    v_ref[...] = jnp.dot(x, wv_ref[...], preferred_element_type=jnp.float32)
    m_ref[...] = jnp.dot(x, wn_ref[...], preferred_element_type=jnp.float32)


def _node_proj(atom, wkt, wvt, wnt):
    n = atom.shape[0]
    grid = (n // _NODE_BLOCK,)
    blk = pl.BlockSpec((_NODE_BLOCK, H), lambda i: (i, 0))
    wblk = pl.BlockSpec((H, H), lambda i: (0, 0))
    return pl.pallas_call(
        _node_proj_body,
        grid=grid,
        in_specs=[blk, wblk, wblk, wblk],
        out_specs=[blk, blk, blk],
        out_shape=[jax.ShapeDtypeStruct((n, H), jnp.float32)] * 3,
    )(atom, wkt, wvt, wnt)


def _final_body(atom_ref, inner_ref, msg_ref, mol_ref, wot_ref, bo_ref,
                g_ref, b_ref, out_ref):
    i = pl.program_id(0)
    outer = _leaky(jnp.dot(msg_ref[...], wot_ref[...],
                           preferred_element_type=jnp.float32) + bo_ref[...])
    x = atom_ref[...] + inner_ref[...] + outer
    m = x.mean(-1, keepdims=True)
    v = ((x - m) ** 2).mean(-1, keepdims=True)
    ln = (x - m) / jnp.sqrt(v + 1e-5) * g_ref[...] + b_ref[...]
    ids = mol_ref[0, 0, :]
    mol_iota = lax.broadcasted_iota(jnp.int32, (256, _NODE_BLOCK), 0)
    onehot = (ids[None, :] == mol_iota).astype(jnp.float32)
    contrib = jnp.dot(onehot, ln, preferred_element_type=jnp.float32,
                      precision=lax.Precision.HIGHEST)

    @pl.when(i == 0)
    def _():
        out_ref[...] = jnp.zeros_like(out_ref)
    out_ref[...] += contrib


def _final_stage(atom, inner, msg, mol_ids, wot, bo, gamma, beta):
    n = atom.shape[0]
    grid = (n // _NODE_BLOCK,)
    blk = pl.BlockSpec((_NODE_BLOCK, H), lambda i: (i, 0))
    vec = pl.BlockSpec((1, H), lambda i: (0, 0))
    mol3 = mol_ids.reshape(n // _NODE_BLOCK, 1, _NODE_BLOCK)
    return pl.pallas_call(
        _final_body,
        grid=grid,
        in_specs=[blk, blk, blk,
                  pl.BlockSpec((1, 1, _NODE_BLOCK), lambda i: (i, 0, 0)),
                  pl.BlockSpec((H, H), lambda i: (0, 0)),
                  vec, vec, vec],
        out_specs=pl.BlockSpec((256, H), lambda i: (0, 0)),
        out_shape=jax.ShapeDtypeStruct((256, H), jnp.float32),
    )(atom, inner, msg, mol3, wot, bo, gamma, beta)


def _zero_own_slice(acc_sh, zb_v, sid, rows_per_sub, zrows):
    def zb_body(i, _):
        for d in range(8):
            zb_v[i, pl.ds(d * 16, 16)] = jnp.zeros((16,), jnp.float32)
        return 0
    lax.fori_loop(0, zrows, zb_body, 0)
    for k in range(rows_per_sub // zrows):
        pltpu.sync_copy(zb_v, acc_sh.at[pl.ds(sid * rows_per_sub + k * zrows,
                                              zrows)])


def _dump_own_slice(acc_sh, zb_v, out_hbm, cid, sid, rows_per_sub, zrows):
    for k in range(rows_per_sub // zrows):
        r0 = sid * rows_per_sub + k * zrows
        pltpu.sync_copy(acc_sh.at[pl.ds(r0, zrows)], zb_v)
        pltpu.sync_copy(zb_v, out_hbm.at[cid, pl.ds(r0, zrows)])


def _sc_inner_phase(m_hbm, e_hbm, idx_hbm, seg_hbm, out_hbm,
                    idx_v, seg_v, rows_v, ef_v, lsem, gsem, acc_sh,
                    cid, sid, wid, n_units, rows_per_sub, zrows):
    _zero_own_slice(acc_sh, rows_v[0], sid, rows_per_sub, zrows)
    plsc.subcore_barrier()

    n_pairs = n_units // 2
    n_mine = (n_pairs // _NW) + jnp.where(wid < (n_pairs % _NW), 1, 0)

    def issue_l(u, s):
        base = u * _UNIT
        pltpu.async_copy(idx_hbm.at[pl.ds(base, _UNIT)], idx_v[s], lsem[s])
        pltpu.async_copy(seg_hbm.at[pl.ds(base, _UNIT)], seg_v[s], lsem[s])
        pltpu.async_copy(e_hbm.at[pl.ds(base, _UNIT)], ef_v[s], lsem[s])

    def wait_l(s):
        pltpu.make_async_copy(idx_hbm.at[pl.ds(0, _UNIT)], idx_v[s], lsem[s]).wait()
        pltpu.make_async_copy(seg_hbm.at[pl.ds(0, _UNIT)], seg_v[s], lsem[s]).wait()
        pltpu.make_async_copy(e_hbm.at[pl.ds(0, _UNIT)], ef_v[s], lsem[s]).wait()

    def issue_g(s):
        pltpu.async_copy(m_hbm.at[idx_v[s]], rows_v[s], gsem[s])

    def wait_g(s):
        pltpu.make_async_copy(m_hbm.at[pl.ds(0, _UNIT)], rows_v[s], gsem[s]).wait()

    def compute(s):
        def mul_body(i, _):
            for d in range(8):
                sl = pl.ds(d * 16, 16)
                rows_v[s][i, sl] = rows_v[s][i, sl] * ef_v[s][i, sl]
            return 0
        lax.fori_loop(0, _UNIT, mul_body, 0)
        pltpu.sync_copy(rows_v[s], acc_sh.at[seg_v[s]], add=True)

    a0 = 2 * wid
    issue_l(a0, 0)
    issue_l(a0 + 1, 1)
    wait_l(0)
    issue_g(0)

    def pair_body(q, _):
        nxt = jnp.minimum(wid + (q + 1) * _NW, n_pairs - 1)
        a2 = 2 * nxt
        wait_l(1)
        issue_g(1)
        wait_g(0)
        compute(0)
        issue_l(a2, 0)
        wait_g(1)
        compute(1)
        issue_l(a2 + 1, 1)
        wait_l(0)
        issue_g(0)
        return 0

    lax.fori_loop(0, n_mine, pair_body, 0)
    wait_g(0)
    wait_l(1)
    plsc.subcore_barrier()
    _dump_own_slice(acc_sh, rows_v[0], out_hbm, cid, sid, rows_per_sub, zrows)
    plsc.subcore_barrier()


_NPAD = 10240


def _sc_inner(m_l, e_l, idx_l, seg_l, m_r, e_r, idx_r, seg_r):
    n_edges = e_l.shape[0]
    n_units = n_edges // _UNIT
    rows_per_sub = _NPAD // _NS
    zrows = _UNIT

    mesh = plsc.VectorSubcoreMesh(core_axis_name="c", subcore_axis_name="s")

    @functools.partial(
        pl.kernel,
        mesh=mesh,
        out_type=(jax.ShapeDtypeStruct((_NC, _NPAD, H), jnp.float32),
                  jax.ShapeDtypeStruct((_NC, _NPAD, H), jnp.float32)),
        scratch_types=[
            [pltpu.VMEM((_UNIT,), jnp.int32)] * 2,
            [pltpu.VMEM((_UNIT,), jnp.int32)] * 2,
            [pltpu.VMEM((_UNIT, H), jnp.float32)] * 2,
            [pltpu.VMEM((_UNIT, H), jnp.float32)] * 2,
            pltpu.VMEM_SHARED((_NPAD, H), jnp.float32),
            [pltpu.SemaphoreType.DMA] * 2,
            [pltpu.SemaphoreType.DMA] * 2,
        ],
        compiler_params=pltpu.CompilerParams(needs_layout_passes=False),
    )
    def k(ml_hbm, el_hbm, il_hbm, sl_hbm, mr_hbm, er_hbm, ir_hbm, sr_hbm,
          outl_hbm, outr_hbm, idx_v, seg_v, rows_v, ef_v, acc_sh, lsem, gsem):
        cid = lax.axis_index("c")
        sid = lax.axis_index("s")
        wid = sid * _NC + cid
        _sc_inner_phase(ml_hbm, el_hbm, il_hbm, sl_hbm, outl_hbm,
                        idx_v, seg_v, rows_v, ef_v, lsem, gsem, acc_sh,
                        cid, sid, wid, n_units, rows_per_sub, zrows)
        _sc_inner_phase(mr_hbm, er_hbm, ir_hbm, sr_hbm, outr_hbm,
                        idx_v, seg_v, rows_v, ef_v, lsem, gsem, acc_sh,
                        cid, sid, wid, n_units, rows_per_sub, zrows)

    return k(m_l, e_l, idx_l, seg_l, m_r, e_r, idx_r, seg_r)


def _mul_by_w(rows_ref, wbuf_ref):
    lanes = lax.iota(jnp.int32, 16)

    def mul_body(e, _):
        wv = wbuf_ref[pl.ds((e // 16) * 16, 16)]
        w = jnp.full((16,), jnp.sum(jnp.where(lanes == e % 16, wv, 0.0)),
                     jnp.float32)
        for d in range(8):
            sl = pl.ds(d * 16, 16)
            rows_ref[e, sl] = rows_ref[e, sl] * w
        return 0
    lax.fori_loop(0, _UNIT, mul_body, 0)


def _sc_att_phase_l(kl_hbm, kr_hbm, vr_hbm, osl_hbm, osr_hbm, w_hbm,
                    ia_v, ib_v, a_v, b_v, c_v, wbuf_v, acc_sh, s_sh,
                    lsem, gsem, wid, n_units):
    inv_t = 1.0 / TEMP
    n_pairs = n_units // 2
    n_mine = (n_pairs // _NW) + jnp.where(wid < (n_pairs % _NW), 1, 0)

    def issue_l(u, s):
        base = u * _UNIT
        pltpu.async_copy(osl_hbm.at[pl.ds(base, _UNIT)], ia_v[s], lsem[s])
        pltpu.async_copy(osr_hbm.at[pl.ds(base, _UNIT)], ib_v[s], lsem[s])

    def wait_l(s):
        pltpu.make_async_copy(osl_hbm.at[pl.ds(0, _UNIT)], ia_v[s], lsem[s]).wait()
        pltpu.make_async_copy(osr_hbm.at[pl.ds(0, _UNIT)], ib_v[s], lsem[s]).wait()

    def issue_g(s):
        pltpu.async_copy(kl_hbm.at[ia_v[s]], a_v[s], gsem[s])
        pltpu.async_copy(kr_hbm.at[ib_v[s]], b_v[s], gsem[s])

    def wait_g(s):
        pltpu.make_async_copy(kl_hbm.at[pl.ds(0, _UNIT)], a_v[s], gsem[s]).wait()
        pltpu.make_async_copy(kr_hbm.at[pl.ds(0, _UNIT)], b_v[s], gsem[s]).wait()

    lanes = lax.iota(jnp.int32, 16)

    def compute(u, s):
        base = u * _UNIT
        gv = pltpu.async_copy(vr_hbm.at[ib_v[s]], c_v, gsem[s])

        def dot_body(e, _):
            acc = a_v[s][e, pl.ds(0, 16)] * b_v[s][e, pl.ds(0, 16)]
            for d in range(1, 8):
                sl = pl.ds(d * 16, 16)
                acc = acc + a_v[s][e, sl] * b_v[s][e, sl]
            wsl = pl.ds((e // 16) * 16, 16)
            wbuf_v[s][wsl] = jnp.where(lanes == e % 16, jnp.sum(acc),
                                       wbuf_v[s][wsl])
            return 0
        lax.fori_loop(0, _UNIT, dot_body, 0)

        def exp_body(g, _):
            sl = pl.ds(g * 16, 16)
            wbuf_v[s][sl] = jnp.exp(wbuf_v[s][sl] * inv_t)
            return 0
        lax.fori_loop(0, _UNIT // 16, exp_body, 0)
        pltpu.sync_copy(wbuf_v[s], w_hbm.at[pl.ds(base, _UNIT)])
        gv.wait()
        _mul_by_w(c_v, wbuf_v[s])
        pltpu.sync_copy(c_v, acc_sh.at[ia_v[s]], add=True)
        pltpu.sync_copy(wbuf_v[s], s_sh.at[ia_v[s]], add=True)

    a0 = 2 * wid
    issue_l(a0, 0)
    issue_l(a0 + 1, 1)
    wait_l(0)
    issue_g(0)

    def pair_body(q, _):
        a = 2 * (wid + q * _NW)
        a2 = 2 * jnp.minimum(wid + (q + 1) * _NW, n_pairs - 1)
        wait_l(1)
        issue_g(1)
        wait_g(0)
        compute(a, 0)
        issue_l(a2, 0)
        wait_g(1)
        compute(a + 1, 1)
        issue_l(a2 + 1, 1)
        wait_l(0)
        issue_g(0)
        return 0

    lax.fori_loop(0, n_mine, pair_body, 0)
    wait_g(0)
    wait_l(1)


def _sc_att_phase_r(vl_hbm, osl_hbm, osr_hbm, w_hbm,
                    ia_v, ib_v, rows2_v, wbuf_v, acc_sh, s_sh,
                    lsem, gsem, wid, n_units):
    n_pairs = n_units // 2
    n_mine = (n_pairs // _NW) + jnp.where(wid < (n_pairs % _NW), 1, 0)

    def issue_l(u, s):
        base = u * _UNIT
        pltpu.async_copy(osl_hbm.at[pl.ds(base, _UNIT)], ia_v[s], lsem[s])
        pltpu.async_copy(osr_hbm.at[pl.ds(base, _UNIT)], ib_v[s], lsem[s])
        pltpu.async_copy(w_hbm.at[pl.ds(base, _UNIT)], wbuf_v[s], lsem[s])

    def wait_l(s):
        pltpu.make_async_copy(osl_hbm.at[pl.ds(0, _UNIT)], ia_v[s], lsem[s]).wait()
        pltpu.make_async_copy(osr_hbm.at[pl.ds(0, _UNIT)], ib_v[s], lsem[s]).wait()
        pltpu.make_async_copy(w_hbm.at[pl.ds(0, _UNIT)], wbuf_v[s], lsem[s]).wait()

    def issue_g(s):
        pltpu.async_copy(vl_hbm.at[ia_v[s]], rows2_v[s], gsem[s])

    def wait_g(s):
        pltpu.make_async_copy(vl_hbm.at[pl.ds(0, _UNIT)], rows2_v[s], gsem[s]).wait()

    def compute(s):
        _mul_by_w(rows2_v[s], wbuf_v[s])
        pltpu.sync_copy(rows2_v[s], acc_sh.at[ib_v[s]], add=True)
        pltpu.sync_copy(wbuf_v[s], s_sh.at[ib_v[s]], add=True)

    a0 = 2 * wid
    issue_l(a0, 0)
    issue_l(a0 + 1, 1)
    wait_l(0)
    issue_g(0)

    def pair_body(q, _):
        a2 = 2 * jnp.minimum(wid + (q + 1) * _NW, n_pairs - 1)
        wait_l(1)
        issue_g(1)
        wait_g(0)
        compute(0)
        issue_l(a2, 0)
        wait_g(1)
        compute(1)
        issue_l(a2 + 1, 1)
        wait_l(0)
        issue_g(0)
        return 0

    lax.fori_loop(0, n_mine, pair_body, 0)
    wait_g(0)
    wait_l(1)


def _zero_s(s_sh, szb_v, sid, spr):
    def z_body(i, _):
        szb_v[pl.ds(i * 16, 16)] = jnp.zeros((16,), jnp.float32)
        return 0
    lax.fori_loop(0, spr // 16, z_body, 0)
    pltpu.sync_copy(szb_v, s_sh.at[pl.ds(sid * spr, spr)])


def _dump_s(s_sh, szb_v, s_hbm, cid, sid, spr):
    pltpu.sync_copy(s_sh.at[pl.ds(sid * spr, spr)], szb_v)
    pltpu.sync_copy(szb_v, s_hbm.at[pl.ds(cid * _NPAD + sid * spr, spr)])


def _sc_attention_l(k_l, k_r, v_r, osl, osr):
    n_edges = osl.shape[0]
    n_units = n_edges // _UNIT
    rows_per_sub = _NPAD // _NS
    spr = _NPAD // _NS

    mesh = plsc.VectorSubcoreMesh(core_axis_name="c", subcore_axis_name="s")

    @functools.partial(
        pl.kernel,
        mesh=mesh,
        out_type=(jax.ShapeDtypeStruct((_NC, _NPAD, H), jnp.float32),
                  jax.ShapeDtypeStruct((_NC * _NPAD,), jnp.float32),
                  jax.ShapeDtypeStruct((n_edges,), jnp.float32)),
        scratch_types=[
            [pltpu.VMEM((_UNIT,), jnp.int32)] * 2,
            [pltpu.VMEM((_UNIT,), jnp.int32)] * 2,
            [pltpu.VMEM((_UNIT, H), jnp.float32)] * 2,
            [pltpu.VMEM((_UNIT, H), jnp.float32)] * 2,
            pltpu.VMEM((_UNIT, H), jnp.float32),
            [pltpu.VMEM((_UNIT,), jnp.float32)] * 2,
            pltpu.VMEM((spr,), jnp.float32),
            pltpu.VMEM_SHARED((_NPAD, H), jnp.float32),
            pltpu.VMEM_SHARED((_NPAD,), jnp.float32),
            [pltpu.SemaphoreType.DMA] * 2,
            [pltpu.SemaphoreType.DMA] * 2,
        ],
        compiler_params=pltpu.CompilerParams(needs_layout_passes=False),
    )
    def k(kl_hbm, kr_hbm, vr_hbm, osl_hbm, osr_hbm,
          accl_hbm, sl_hbm, w_hbm,
          ia_v, ib_v, a_v, b_v, c_v, wbuf_v, szb_v, acc_sh, s_sh, lsem, gsem):
        cid = lax.axis_index("c")
        sid = lax.axis_index("s")
        wid = sid * _NC + cid
        _zero_own_slice(acc_sh, a_v[0], sid, rows_per_sub, _UNIT)
        _zero_s(s_sh, szb_v, sid, spr)
        plsc.subcore_barrier()
        _sc_att_phase_l(kl_hbm, kr_hbm, vr_hbm, osl_hbm, osr_hbm, w_hbm,
                        ia_v, ib_v, a_v, b_v, c_v, wbuf_v, acc_sh, s_sh,
                        lsem, gsem, wid, n_units)
        plsc.subcore_barrier()
        _dump_own_slice(acc_sh, a_v[0], accl_hbm, cid, sid, rows_per_sub, _UNIT)
        _dump_s(s_sh, szb_v, sl_hbm, cid, sid, spr)

    return k(k_l, k_r, v_r, osl, osr)


def _sc_attention_r(v_l, osl, osr, w):
    n_edges = osl.shape[0]
    n_units = n_edges // _UNIT
    rows_per_sub = _NPAD // _NS
    spr = _NPAD // _NS

    mesh = plsc.VectorSubcoreMesh(core_axis_name="c", subcore_axis_name="s")

    @functools.partial(
        pl.kernel,
        mesh=mesh,
        out_type=(jax.ShapeDtypeStruct((_NC, _NPAD, H), jnp.float32),
                  jax.ShapeDtypeStruct((_NC * _NPAD,), jnp.float32)),
        scratch_types=[
            [pltpu.VMEM((_UNIT,), jnp.int32)] * 2,
            [pltpu.VMEM((_UNIT,), jnp.int32)] * 2,
            [pltpu.VMEM((_UNIT, H), jnp.float32)] * 2,
            [pltpu.VMEM((_UNIT,), jnp.float32)] * 2,
            pltpu.VMEM((spr,), jnp.float32),
            pltpu.VMEM_SHARED((_NPAD, H), jnp.float32),
            pltpu.VMEM_SHARED((_NPAD,), jnp.float32),
            [pltpu.SemaphoreType.DMA] * 2,
            [pltpu.SemaphoreType.DMA] * 2,
        ],
        compiler_params=pltpu.CompilerParams(needs_layout_passes=False),
    )
    def k(vl_hbm, osl_hbm, osr_hbm, w_hbm, accr_hbm, sr_hbm,
          ia_v, ib_v, r_v, wbuf_v, szb_v, acc_sh, s_sh, lsem, gsem):
        cid = lax.axis_index("c")
        sid = lax.axis_index("s")
        wid = sid * _NC + cid
        _zero_own_slice(acc_sh, r_v[0], sid, rows_per_sub, _UNIT)
        _zero_s(s_sh, szb_v, sid, spr)
        plsc.subcore_barrier()
        _sc_att_phase_r(vl_hbm, osl_hbm, osr_hbm, w_hbm,
                        ia_v, ib_v, r_v, wbuf_v, acc_sh, s_sh,
                        lsem, gsem, wid, n_units)
        plsc.subcore_barrier()
        _dump_own_slice(acc_sh, r_v[0], accr_hbm, cid, sid, rows_per_sub, _UNIT)
        _dump_s(s_sh, szb_v, sr_hbm, cid, sid, spr)

    return k(v_l, osl, osr, w)


def kernel(segmentation_molecule_left, atom_left, bond_left,
           inner_segmentation_index_left, inner_index_left,
           outer_segmentation_index_left, outer_index_left,
           segmentation_molecule_right, atom_right, bond_right,
           inner_segmentation_index_right, inner_index_right,
           outer_segmentation_index_right, outer_index_right,
           mp_node_W, mp_edge_W1, mp_edge_b1, mp_edge_W2, mp_edge_b2,
           att_key_W, att_value_W, att_out_W, att_out_b, ln_gamma, ln_beta):
    n_nodes = atom_left.shape[0]
    n_mols = 256

    w1t = mp_edge_W1.T
    w2t = mp_edge_W2.T
    b1 = mp_edge_b1[None, :]
    b2 = mp_edge_b2[None, :]

    e_l = _edge_proj(bond_left, w1t, b1, w2t, b2)
    e_r = _edge_proj(bond_right, w1t, b1, w2t, b2)

    k_l, v_l, m_l = _node_proj(atom_left, att_key_W.T, att_value_W.T,
                               mp_node_W.T)
    k_r, v_r, m_r = _node_proj(atom_right, att_key_W.T, att_value_W.T,
                               mp_node_W.T)

    osl = outer_segmentation_index_left
    osr = outer_segmentation_index_right
    pal, psl, w_e = _sc_attention_l(k_l, k_r, v_r, osl, osr)
    par, psr = _sc_attention_r(v_l, osl, osr, w_e)
    ml_msg = ((pal[0] + pal[1])[:n_nodes]
              / (psl[:_NPAD] + psl[_NPAD:] + 1e-16)[:n_nodes, None])
    mr_msg = ((par[0] + par[1])[:n_nodes]
              / (psr[:_NPAD] + psr[_NPAD:] + 1e-16)[:n_nodes, None])

    pin_l, pin_r = _sc_inner(
        m_l, e_l, inner_index_left, inner_segmentation_index_left,
        m_r, e_r, inner_index_right, inner_segmentation_index_right)
    inner_l = (pin_l[0] + pin_l[1])[:n_nodes]
    inner_r = (pin_r[0] + pin_r[1])[:n_nodes]

    graph_l = _final_stage(atom_left, inner_l, ml_msg,
                           segmentation_molecule_left, att_out_W.T,
                           att_out_b[None, :], ln_gamma[None, :],
                           ln_beta[None, :])
    graph_r = _final_stage(atom_right, inner_r, mr_msg,
                           segmentation_molecule_right, att_out_W.T,
                           att_out_b[None, :], ln_gamma[None, :],
                           ln_beta[None, :])
    return (graph_l, graph_r)
